# Initial kernel scaffold; baseline (speedup 1.0000x reference)
import jax
import jax.numpy as jnp
from jax.experimental import pallas as pl


def kernel(node_emb, rel_emb, head_indices, rel_types, tail_indices, sim_scores, neighbor_idx, degrees):
    raise NotImplementedError("write your pallas kernel here")


# speedup vs baseline: 2.1197x; 2.1197x over previous
import functools

import jax
import jax.numpy as jnp
from jax import lax
from jax.experimental import pallas as pl
from jax.experimental.pallas import tpu as pltpu
from jax.experimental.pallas import tpu_sc as plsc

N_NODES = 1_000_000
NUM_RELS = 32
D = 64
B = 16384
K = 10
L = 16
NC, NS = 2, 16
NW = NC * NS
BPW = B // NW
SUB = 64
NSUB = BPW // SUB
R_PAD = 31256
OWNER_SZ = NW * R_PAD


def _iota16():
    return lax.iota(jnp.int32, L)


def _owner_build(head_full, owner_sl, base):
    def body(g, _):
        hv = head_full[pl.ds(g * L, L)]
        j = g * L + _iota16()
        local = hv - base
        m = (local >= 0) & (local < R_PAD)
        lc = jnp.minimum(jnp.maximum(local, 0), R_PAD - 1)
        plsc.store_scatter(owner_sl, [lc], j, mask=m)
        g2 = plsc.load_gather(owner_sl, [lc], mask=m)
        plsc.store_scatter(owner_sl, [lc], j, mask=m & (g2 < j))
        g3 = plsc.load_gather(owner_sl, [lc], mask=m)
        plsc.store_scatter(owner_sl, [lc], j, mask=m & (g3 < j))
        return 0
    lax.fori_loop(0, B // L, body, 0)


def _phase1(node_emb, head_idx, nidx2, sim, rel_t, deg, owner_hbm, upd_hbm,
            head_full, owner_sl, nidx_v, neigh_v, hrows_v, sim_v, rel_v,
            deg_v, upd_v, sem):
    wid = lax.axis_index("c") * NS + lax.axis_index("s")
    pltpu.sync_copy(head_idx, head_full)
    base = wid * R_PAD
    _owner_build(head_full, owner_sl, base)
    pltpu.sync_copy(owner_sl, owner_hbm.at[pl.ds(base, R_PAD)])
    pltpu.sync_copy(nidx2.at[pl.ds(wid * (BPW * K // 128), BPW * K // 128)],
                    nidx_v)

    def sub(s, _):
        rowbase = wid * BPW + s * SUB
        pltpu.sync_copy(sim.at[pl.ds(rowbase, SUB)], sim_v)
        pltpu.sync_copy(rel_t.at[pl.ds(rowbase, SUB)], rel_v)
        pltpu.sync_copy(deg.at[pl.ds(rowbase, SUB)], deg_v)
        cps = []
        for jj in range(SUB * K // 128):
            cps.append(pltpu.async_copy(
                node_emb.at[nidx_v.at[s * (SUB * K // 128) + jj]],
                neigh_v.at[pl.ds(jj * 128, 128)], sem))
        cps.append(pltpu.async_copy(
            node_emb.at[head_full.at[pl.ds(rowbase, SUB)]], hrows_v, sem))
        for cp in cps:
            cp.wait()
        for g2 in range(SUB // L):
            row = g2 * L + _iota16()
            rl = rel_v[pl.ds(g2 * L, L)]
            dg = deg_v[pl.ds(g2 * L, L)]
            msk = (rl >= 2) & (rl <= 4)
            c = 0.7 * jnp.exp(-0.7 * dg.astype(jnp.float32)) + 0.2
            sk = [plsc.load_gather(sim_v, [row, jnp.full((L,), k, jnp.int32)])
                  for k in range(K)]
            nrow = [row * K + k for k in range(K)]

            def dbody(d, _, row=row, sk=sk, nrow=nrow, msk=msk, c=c):
                dv = jnp.full((L,), d, jnp.int32)
                acc = jnp.zeros((L,), jnp.float32)
                for k in range(K):
                    acc += sk[k] * plsc.load_gather(neigh_v, [nrow[k], dv])
                oh = plsc.load_gather(hrows_v, [row, dv])
                out = jnp.where(msk, c * acc + (1.0 - c) * oh, oh)
                plsc.store_scatter(upd_v, [row, dv], out)
                return 0
            lax.fori_loop(0, D, dbody, 0)
        pltpu.sync_copy(upd_v, upd_hbm.at[pl.ds(rowbase, SUB)])
        return 0
    lax.fori_loop(0, NSUB, sub, 0)


def _phase2(node_emb, rel_emb, head_idx, tail_idx, rel_t, owner_hbm, upd_hbm,
            score_hbm, head_full, hidx_v, tidx_v, rel_v, wh_v, wt_v, wtc_v,
            val_v, hrows_v, turow_v, torow_v, relv_v, scores_v, sem):
    wid = lax.axis_index("c") * NS + lax.axis_index("s")
    pltpu.sync_copy(head_idx, head_full)
    pltpu.sync_copy(rel_emb, relv_v)

    def sub(s, _):
        rowbase = wid * BPW + s * SUB
        pltpu.sync_copy(head_idx.at[pl.ds(rowbase, SUB)], hidx_v)
        pltpu.sync_copy(tail_idx.at[pl.ds(rowbase, SUB)], tidx_v)
        pltpu.sync_copy(rel_t.at[pl.ds(rowbase, SUB)], rel_v)
        c1 = pltpu.async_copy(owner_hbm.at[hidx_v], wh_v, sem)
        c2 = pltpu.async_copy(owner_hbm.at[tidx_v], wt_v, sem)
        c1.wait()
        c2.wait()
        for g2 in range(SUB // L):
            sl = pl.ds(g2 * L, L)
            wt = wt_v[sl]
            wtc = jnp.minimum(jnp.maximum(wt, 0), B - 1)
            hit = plsc.load_gather(head_full, [wtc])
            ok = (wt >= 0) & (wt < B) & (hit == tidx_v[sl])
            wtc_v[sl] = wtc
            val_v[sl] = jnp.where(ok, 1.0, 0.0)
        c3 = pltpu.async_copy(upd_hbm.at[wh_v], hrows_v, sem)
        c4 = pltpu.async_copy(upd_hbm.at[wtc_v], turow_v, sem)
        c5 = pltpu.async_copy(node_emb.at[tidx_v], torow_v, sem)
        c3.wait()
        c4.wait()
        c5.wait()
        for g2 in range(SUB // L):
            row = g2 * L + _iota16()
            rl = rel_v[pl.ds(g2 * L, L)]
            vd = val_v[pl.ds(g2 * L, L)] > 0.5

            def dbody(d, acc, row=row, rl=rl, vd=vd):
                dv = jnp.full((L,), d, jnp.int32)
                h = plsc.load_gather(hrows_v, [row, dv])
                tu = plsc.load_gather(turow_v, [row, dv])
                to = plsc.load_gather(torow_v, [row, dv])
                r = plsc.load_gather(relv_v, [rl, dv])
                return acc + h * r * jnp.where(vd, tu, to)
            acc = lax.fori_loop(0, D, dbody, jnp.zeros((L,), jnp.float32))
            sig = 1.0 / (1.0 + jnp.exp(-acc))
            scores_v[pl.ds(s * SUB + g2 * L, L)] = sig
        return 0
    lax.fori_loop(0, NSUB, sub, 0)
    pltpu.sync_copy(scores_v, score_hbm.at[pl.ds(wid * BPW, BPW)])


def kernel(node_emb, rel_emb, head_indices, rel_types, tail_indices,
           sim_scores, neighbor_idx, degrees):
    head_i = head_indices.astype(jnp.int32)
    tail_i = tail_indices.astype(jnp.int32)
    rel_i = rel_types.astype(jnp.int32)
    deg_i = degrees.astype(jnp.int32)
    nidx2 = neighbor_idx.astype(jnp.int32).reshape(B * K // 128, 128)

    mesh = plsc.VectorSubcoreMesh(core_axis_name="c", subcore_axis_name="s")

    k1 = pl.kernel(
        _phase1,
        out_type=(jax.ShapeDtypeStruct((OWNER_SZ,), jnp.int32),
                  jax.ShapeDtypeStruct((B, D), jnp.float32)),
        mesh=mesh,
        compiler_params=pltpu.CompilerParams(needs_layout_passes=False, use_tc_tiling_on_sc=False),
        scratch_types=[
            pltpu.VMEM((B,), jnp.int32),
            pltpu.VMEM((R_PAD,), jnp.int32),
            pltpu.VMEM((BPW * K // 128, 128), jnp.int32),
            pltpu.VMEM((SUB * K, D), jnp.float32),
            pltpu.VMEM((SUB, D), jnp.float32),
            pltpu.VMEM((SUB, K), jnp.float32),
            pltpu.VMEM((SUB,), jnp.int32),
            pltpu.VMEM((SUB,), jnp.int32),
            pltpu.VMEM((SUB, D), jnp.float32),
            pltpu.SemaphoreType.DMA,
        ],
    )
    owner, upd = k1(node_emb, head_i, nidx2, sim_scores, rel_i, deg_i)

    k2 = pl.kernel(
        _phase2,
        out_type=jax.ShapeDtypeStruct((B,), jnp.float32),
        mesh=mesh,
        compiler_params=pltpu.CompilerParams(needs_layout_passes=False, use_tc_tiling_on_sc=False),
        scratch_types=[
            pltpu.VMEM((B,), jnp.int32),
            pltpu.VMEM((SUB,), jnp.int32),
            pltpu.VMEM((SUB,), jnp.int32),
            pltpu.VMEM((SUB,), jnp.int32),
            pltpu.VMEM((SUB,), jnp.int32),
            pltpu.VMEM((SUB,), jnp.int32),
            pltpu.VMEM((SUB,), jnp.int32),
            pltpu.VMEM((SUB,), jnp.float32),
            pltpu.VMEM((SUB, D), jnp.float32),
            pltpu.VMEM((SUB, D), jnp.float32),
            pltpu.VMEM((SUB, D), jnp.float32),
            pltpu.VMEM((NUM_RELS, D), jnp.float32),
            pltpu.VMEM((BPW,), jnp.float32),
            pltpu.SemaphoreType.DMA,
        ],
    )
    return k2(node_emb, rel_emb, head_i, tail_i, rel_i, owner, upd)


# speedup vs baseline: 2.8894x; 1.3631x over previous
import jax
import jax.numpy as jnp
from jax import lax
from jax.experimental import pallas as pl
from jax.experimental.pallas import tpu as pltpu
from jax.experimental.pallas import tpu_sc as plsc

N_NODES = 1_000_000
NUM_RELS = 32
D = 64
B = 16384
K = 10
KP = 16
L = 16
NC, NS = 2, 16
NW = NC * NS
BPW = B // NW
SUB = 128
NSUB = BPW // SUB
SL = 62512
EB = 16
MAXE = 2 * SUB


def _iota16():
    return lax.iota(jnp.int32, L)


def _body(node_emb, rel_emb, head_idx, tail_idx, rel_t, deg, sim_p, nid_p,
          score_hbm, owner_sh, head_full, owner_sl, relv_v, hidx_v, tidx_v,
          rl_v, wh_v, wt_v, wtc_v, relw_v, went_v, qref_v, eidx_v, sim_e,
          nid_e, deg_e, nflat_v, neigh_v, hn_v, tn_v, scores_v, sem):
    cid = lax.axis_index("c")
    sid = lax.axis_index("s")
    wid = cid * NS + sid
    pltpu.sync_copy(head_idx, head_full)
    pltpu.sync_copy(rel_emb, relv_v)

    base = sid * SL

    def oscan(g, _):
        hv = head_full[pl.ds(g * L, L)]
        j = g * L + _iota16()
        local = hv - base
        m = (local >= 0) & (local < SL)
        lc = jnp.minimum(jnp.maximum(local, 0), SL - 1)
        plsc.store_scatter(owner_sl, [lc], j, mask=m)
        g2 = plsc.load_gather(owner_sl, [lc], mask=m)
        plsc.store_scatter(owner_sl, [lc], j, mask=m & (g2 < j))
        g3 = plsc.load_gather(owner_sl, [lc], mask=m)
        plsc.store_scatter(owner_sl, [lc], j, mask=m & (g3 < j))
        return 0
    lax.fori_loop(0, B // L, oscan, 0)
    pltpu.sync_copy(owner_sl, owner_sh.at[pl.ds(base, SL)])
    plsc.subcore_barrier()

    def sub(s, _):
        rowbase = wid * BPW + s * SUB
        pltpu.sync_copy(head_idx.at[pl.ds(rowbase, SUB)], hidx_v)
        pltpu.sync_copy(tail_idx.at[pl.ds(rowbase, SUB)], tidx_v)
        pltpu.sync_copy(rel_t.at[pl.ds(rowbase, SUB)], rl_v)
        c1 = pltpu.async_copy(owner_sh.at[hidx_v], wh_v, sem)
        c2 = pltpu.async_copy(owner_sh.at[tidx_v], wt_v, sem)
        c3 = pltpu.async_copy(node_emb.at[hidx_v], hn_v, sem)
        c4 = pltpu.async_copy(node_emb.at[tidx_v], tn_v, sem)
        c1.wait()
        c2.wait()
        c3.wait()
        c4.wait()
        for g in range(SUB // L):
            sl_ = pl.ds(g * L, L)
            wt = wt_v[sl_]
            wtc_v[sl_] = jnp.minimum(jnp.maximum(wt, 0), B - 1)
        c5 = pltpu.async_copy(rel_t.at[wh_v], relw_v, sem)
        c6 = pltpu.async_copy(rel_t.at[wtc_v], wt_v, sem)
        c5.wait()
        c6.wait()

        def centry(g, cnt):
            sl_ = pl.ds(g * L, L)
            qpos = g * L + _iota16()
            mh = (relw_v[sl_] >= 2) & (relw_v[sl_] <= 4)
            pos = cnt + plsc.cumsum(mh.astype(jnp.int32)) - 1
            plsc.store_scatter(went_v, [pos], wh_v[sl_], mask=mh)
            plsc.store_scatter(qref_v, [pos], qpos, mask=mh)
            cnt = cnt + jnp.sum(mh.astype(jnp.int32))
            wt0 = plsc.load_gather(head_full, [wtc_v[sl_]])
            ok = (wt_v[sl_] >= 2) & (wt_v[sl_] <= 4) & (wt0 == tidx_v[sl_])
            pos = cnt + plsc.cumsum(ok.astype(jnp.int32)) - 1
            plsc.store_scatter(went_v, [pos], wtc_v[sl_], mask=ok)
            plsc.store_scatter(qref_v, [pos], qpos + SUB, mask=ok)
            return cnt + jnp.sum(ok.astype(jnp.int32))
        nent = lax.fori_loop(0, SUB // L, centry, jnp.int32(0))

        def eblock(b, _):
            ei = b * EB + _iota16()
            elane = ei < nent
            eic = jnp.minimum(ei, MAXE - 1)
            went = plsc.load_gather(went_v, [eic], mask=elane)
            went = jnp.where(elane, went, _iota16())
            qref = plsc.load_gather(qref_v, [eic], mask=elane)
            qref = jnp.where(elane, qref, 0)
            eidx_v[pl.ds(0, L)] = went
            ce1 = pltpu.async_copy(sim_p.at[eidx_v], sim_e, sem)
            ce2 = pltpu.async_copy(nid_p.at[eidx_v], nid_e, sem)
            ce3 = pltpu.async_copy(deg.at[eidx_v], deg_e, sem)
            ce1.wait()
            ce2.wait()
            ce3.wait()
            for k in range(K):
                ids = plsc.load_gather(nid_e, [_iota16(),
                                               jnp.full((L,), k, jnp.int32)])
                ids = jnp.where(elane, ids, _iota16() * 8 + k)
                plsc.store_scatter(nflat_v, [_iota16() * K + k], ids)
            cn1 = pltpu.async_copy(node_emb.at[nflat_v.at[pl.ds(0, 128)]],
                                   neigh_v.at[pl.ds(0, 128)], sem)
            cn2 = pltpu.async_copy(node_emb.at[nflat_v.at[pl.ds(128, 32)]],
                                   neigh_v.at[pl.ds(128, 32)], sem)
            dge = deg_e[pl.ds(0, L)]
            cc = 0.7 * jnp.exp(-0.7 * dge.astype(jnp.float32)) + 0.2
            cn1.wait()
            cn2.wait()
            sk = [plsc.load_gather(sim_e, [_iota16(),
                                           jnp.full((L,), k, jnp.int32)])
                  for k in range(K)]
            ishead = qref < SUB
            qp = jnp.minimum(qref, SUB - 1)
            qp2 = jnp.minimum(jnp.maximum(qref - SUB, 0), SUB - 1)

            def dbody(d, _, sk=sk, cc=cc, ishead=ishead, qp=qp, qp2=qp2,
                      elane=elane):
                dv = jnp.full((L,), d, jnp.int32)
                acc = jnp.zeros((L,), jnp.float32)
                for k in range(K):
                    acc += sk[k] * plsc.load_gather(
                        neigh_v, [_iota16() * K + k, dv])
                oh = jnp.where(ishead,
                               plsc.load_gather(hn_v, [qp, dv]),
                               plsc.load_gather(tn_v, [qp2, dv]))
                val = cc * acc + (1.0 - cc) * oh
                plsc.store_scatter(hn_v, [qp, dv], val, mask=ishead & elane)
                plsc.store_scatter(tn_v, [qp2, dv], val,
                                   mask=(~ishead) & elane)
                return 0
            lax.fori_loop(0, D, dbody, 0)
            return 0
        nblk = (nent + EB - 1) // EB
        lax.fori_loop(0, nblk, eblock, 0)

        for g in range(SUB // L):
            row = g * L + _iota16()
            rl = rl_v[pl.ds(g * L, L)]

            def sbody(d, acc, row=row, rl=rl):
                dv = jnp.full((L,), d, jnp.int32)
                h = plsc.load_gather(hn_v, [row, dv])
                t = plsc.load_gather(tn_v, [row, dv])
                r = plsc.load_gather(relv_v, [rl, dv])
                return acc + h * r * t
            acc = lax.fori_loop(0, D, sbody, jnp.zeros((L,), jnp.float32))
            scores_v[pl.ds(s * SUB + g * L, L)] = 1.0 / (1.0 + jnp.exp(-acc))
        return 0
    lax.fori_loop(0, NSUB, sub, 0)
    pltpu.sync_copy(scores_v, score_hbm.at[pl.ds(wid * BPW, BPW)])


def kernel(node_emb, rel_emb, head_indices, rel_types, tail_indices,
           sim_scores, neighbor_idx, degrees):
    head_i = head_indices.astype(jnp.int32)
    tail_i = tail_indices.astype(jnp.int32)
    rel_i = rel_types.astype(jnp.int32)
    deg_i = degrees.astype(jnp.int32)
    sim_p = jnp.pad(sim_scores, ((0, 0), (0, KP - K)))
    nid_p = jnp.pad(neighbor_idx.astype(jnp.int32), ((0, 0), (0, KP - K)))

    mesh = plsc.VectorSubcoreMesh(core_axis_name="c", subcore_axis_name="s")
    k = pl.kernel(
        _body,
        out_type=(jax.ShapeDtypeStruct((B,), jnp.float32),
                  jax.ShapeDtypeStruct((NS * SL,), jnp.int32)),
        mesh=mesh,
        compiler_params=pltpu.CompilerParams(
            needs_layout_passes=False, use_tc_tiling_on_sc=False),
        scratch_types=[
            pltpu.VMEM((B,), jnp.int32),
            pltpu.VMEM((SL,), jnp.int32),
            pltpu.VMEM((NUM_RELS, D), jnp.float32),
            pltpu.VMEM((SUB,), jnp.int32),
            pltpu.VMEM((SUB,), jnp.int32),
            pltpu.VMEM((SUB,), jnp.int32),
            pltpu.VMEM((SUB,), jnp.int32),
            pltpu.VMEM((SUB,), jnp.int32),
            pltpu.VMEM((SUB,), jnp.int32),
            pltpu.VMEM((SUB,), jnp.int32),
            pltpu.VMEM((MAXE,), jnp.int32),
            pltpu.VMEM((MAXE,), jnp.int32),
            pltpu.VMEM((L,), jnp.int32),
            pltpu.VMEM((L, KP), jnp.float32),
            pltpu.VMEM((L, KP), jnp.int32),
            pltpu.VMEM((L,), jnp.int32),
            pltpu.VMEM((EB * K,), jnp.int32),
            pltpu.VMEM((EB * K, D), jnp.float32),
            pltpu.VMEM((SUB, D), jnp.float32),
            pltpu.VMEM((SUB, D), jnp.float32),
            pltpu.VMEM((BPW,), jnp.float32),
            pltpu.SemaphoreType.DMA,
        ],
    )
    return k(node_emb, rel_emb, head_i, tail_i, rel_i, deg_i, sim_p, nid_p)[0]


# speedup vs baseline: 2.9101x; 1.0072x over previous
import jax
import jax.numpy as jnp
from jax import lax
from jax.experimental import pallas as pl
from jax.experimental.pallas import tpu as pltpu
from jax.experimental.pallas import tpu_sc as plsc

N_NODES = 1_000_000
NUM_RELS = 32
D = 64
B = 16384
K = 10
KP = 16
L = 16
NC, NS = 2, 16
NW = NC * NS
BPW = B // NW
SUB = 128
NSUB = BPW // SUB
SL = 62512
EB = 16
MAXE = 2 * SUB


def _iota16():
    return lax.iota(jnp.int32, L)


def _body(node_emb, rel_emb, head_idx, tail_idx, rel_t, deg, sim_f, nid_f,
          score_hbm, owner_sh, head_full, owner_sl, relv_v, hidx_v, tidx_v,
          rl_v, wh_v, wt_v, wtc_v, relw_v, went_v, qref_v, eidx_v, kidx_v,
          sims_e, deg_e, nflat_v, neigh_v, hn_v, tn_v, scores_v, sem):
    cid = lax.axis_index("c")
    sid = lax.axis_index("s")
    wid = cid * NS + sid
    pltpu.sync_copy(head_idx, head_full)
    pltpu.sync_copy(rel_emb, relv_v)

    base = sid * SL

    def oscan(g, _):
        hv = head_full[pl.ds(g * L, L)]
        j = g * L + _iota16()
        local = hv - base
        m = (local >= 0) & (local < SL)
        lc = jnp.minimum(jnp.maximum(local, 0), SL - 1)
        plsc.store_scatter(owner_sl, [lc], j, mask=m)
        g2 = plsc.load_gather(owner_sl, [lc], mask=m)
        plsc.store_scatter(owner_sl, [lc], j, mask=m & (g2 < j))
        g3 = plsc.load_gather(owner_sl, [lc], mask=m)
        plsc.store_scatter(owner_sl, [lc], j, mask=m & (g3 < j))
        return 0
    lax.fori_loop(0, B // L, oscan, 0)
    pltpu.sync_copy(owner_sl, owner_sh.at[pl.ds(base, SL)])
    plsc.subcore_barrier()

    def sub(s, _):
        rowbase = wid * BPW + s * SUB
        pltpu.sync_copy(head_idx.at[pl.ds(rowbase, SUB)], hidx_v)
        pltpu.sync_copy(tail_idx.at[pl.ds(rowbase, SUB)], tidx_v)
        pltpu.sync_copy(rel_t.at[pl.ds(rowbase, SUB)], rl_v)
        c1 = pltpu.async_copy(owner_sh.at[hidx_v], wh_v, sem)
        c2 = pltpu.async_copy(owner_sh.at[tidx_v], wt_v, sem)
        c3 = pltpu.async_copy(node_emb.at[hidx_v], hn_v, sem)
        c4 = pltpu.async_copy(node_emb.at[tidx_v], tn_v, sem)
        c1.wait()
        c2.wait()
        c3.wait()
        c4.wait()
        for g in range(SUB // L):
            sl_ = pl.ds(g * L, L)
            wt = wt_v[sl_]
            wtc_v[sl_] = jnp.minimum(jnp.maximum(wt, 0), B - 1)
        c5 = pltpu.async_copy(rel_t.at[wh_v], relw_v, sem)
        c6 = pltpu.async_copy(rel_t.at[wtc_v], wt_v, sem)
        c5.wait()
        c6.wait()

        def centry(g, cnt):
            sl_ = pl.ds(g * L, L)
            qpos = g * L + _iota16()
            mh = (relw_v[sl_] >= 2) & (relw_v[sl_] <= 4)
            pos = cnt + plsc.cumsum(mh.astype(jnp.int32)) - 1
            plsc.store_scatter(went_v, [pos], wh_v[sl_], mask=mh)
            plsc.store_scatter(qref_v, [pos], qpos, mask=mh)
            cnt = cnt + jnp.sum(mh.astype(jnp.int32))
            wt0 = plsc.load_gather(head_full, [wtc_v[sl_]])
            ok = (wt_v[sl_] >= 2) & (wt_v[sl_] <= 4) & (wt0 == tidx_v[sl_])
            pos = cnt + plsc.cumsum(ok.astype(jnp.int32)) - 1
            plsc.store_scatter(went_v, [pos], wtc_v[sl_], mask=ok)
            plsc.store_scatter(qref_v, [pos], qpos + SUB, mask=ok)
            return cnt + jnp.sum(ok.astype(jnp.int32))
        nent = lax.fori_loop(0, SUB // L, centry, jnp.int32(0))

        def eblock(b, _):
            ei = b * EB + _iota16()
            elane = ei < nent
            eic = jnp.minimum(ei, MAXE - 1)
            went = plsc.load_gather(went_v, [eic], mask=elane)
            went = jnp.where(elane, went, _iota16())
            qref = plsc.load_gather(qref_v, [eic], mask=elane)
            qref = jnp.where(elane, qref, 0)
            eidx_v[pl.ds(0, L)] = went
            for k in range(K):
                plsc.store_scatter(kidx_v, [_iota16() * K + k], went * K + k)
            ce1 = pltpu.async_copy(sim_f.at[kidx_v.at[pl.ds(0, 128)]],
                                   sims_e.at[pl.ds(0, 128)], sem)
            ce2 = pltpu.async_copy(sim_f.at[kidx_v.at[pl.ds(128, 32)]],
                                   sims_e.at[pl.ds(128, 32)], sem)
            ce3 = pltpu.async_copy(nid_f.at[kidx_v.at[pl.ds(0, 128)]],
                                   nflat_v.at[pl.ds(0, 128)], sem)
            ce4 = pltpu.async_copy(nid_f.at[kidx_v.at[pl.ds(128, 32)]],
                                   nflat_v.at[pl.ds(128, 32)], sem)
            ce5 = pltpu.async_copy(deg.at[eidx_v], deg_e, sem)
            ce1.wait()
            ce2.wait()
            ce3.wait()
            ce4.wait()
            ce5.wait()
            cn1 = pltpu.async_copy(node_emb.at[nflat_v.at[pl.ds(0, 128)]],
                                   neigh_v.at[pl.ds(0, 128)], sem)
            cn2 = pltpu.async_copy(node_emb.at[nflat_v.at[pl.ds(128, 32)]],
                                   neigh_v.at[pl.ds(128, 32)], sem)
            dge = deg_e[pl.ds(0, L)]
            cc = 0.7 * jnp.exp(-0.7 * dge.astype(jnp.float32)) + 0.2
            cn1.wait()
            cn2.wait()
            sk = [plsc.load_gather(sims_e, [_iota16() * K + k])
                  for k in range(K)]
            ishead = qref < SUB
            qp = jnp.minimum(qref, SUB - 1)
            qp2 = jnp.minimum(jnp.maximum(qref - SUB, 0), SUB - 1)

            def dbody(d, _, sk=sk, cc=cc, ishead=ishead, qp=qp, qp2=qp2,
                      elane=elane):
                dv = jnp.full((L,), d, jnp.int32)
                acc = jnp.zeros((L,), jnp.float32)
                for k in range(K):
                    acc += sk[k] * plsc.load_gather(
                        neigh_v, [_iota16() * K + k, dv])
                oh = jnp.where(ishead,
                               plsc.load_gather(hn_v, [qp, dv]),
                               plsc.load_gather(tn_v, [qp2, dv]))
                val = cc * acc + (1.0 - cc) * oh
                plsc.store_scatter(hn_v, [qp, dv], val, mask=ishead & elane)
                plsc.store_scatter(tn_v, [qp2, dv], val,
                                   mask=(~ishead) & elane)
                return 0
            lax.fori_loop(0, D, dbody, 0)
            return 0
        nblk = (nent + EB - 1) // EB
        lax.fori_loop(0, nblk, eblock, 0)

        for g in range(SUB // L):
            row = g * L + _iota16()
            rl = rl_v[pl.ds(g * L, L)]

            def sbody(d, acc, row=row, rl=rl):
                dv = jnp.full((L,), d, jnp.int32)
                h = plsc.load_gather(hn_v, [row, dv])
                t = plsc.load_gather(tn_v, [row, dv])
                r = plsc.load_gather(relv_v, [rl, dv])
                return acc + h * r * t
            acc = lax.fori_loop(0, D, sbody, jnp.zeros((L,), jnp.float32))
            scores_v[pl.ds(s * SUB + g * L, L)] = 1.0 / (1.0 + jnp.exp(-acc))
        return 0
    lax.fori_loop(0, NSUB, sub, 0)
    pltpu.sync_copy(scores_v, score_hbm.at[pl.ds(wid * BPW, BPW)])


def kernel(node_emb, rel_emb, head_indices, rel_types, tail_indices,
           sim_scores, neighbor_idx, degrees):
    head_i = head_indices.astype(jnp.int32)
    tail_i = tail_indices.astype(jnp.int32)
    rel_i = rel_types.astype(jnp.int32)
    deg_i = degrees.astype(jnp.int32)
    sim_f = sim_scores.reshape(B * K)
    nid_f = neighbor_idx.astype(jnp.int32).reshape(B * K)

    mesh = plsc.VectorSubcoreMesh(core_axis_name="c", subcore_axis_name="s")
    k = pl.kernel(
        _body,
        out_type=(jax.ShapeDtypeStruct((B,), jnp.float32),
                  jax.ShapeDtypeStruct((NS * SL,), jnp.int32)),
        mesh=mesh,
        compiler_params=pltpu.CompilerParams(
            needs_layout_passes=False, use_tc_tiling_on_sc=False),
        scratch_types=[
            pltpu.VMEM((B,), jnp.int32),
            pltpu.VMEM((SL,), jnp.int32),
            pltpu.VMEM((NUM_RELS, D), jnp.float32),
            pltpu.VMEM((SUB,), jnp.int32),
            pltpu.VMEM((SUB,), jnp.int32),
            pltpu.VMEM((SUB,), jnp.int32),
            pltpu.VMEM((SUB,), jnp.int32),
            pltpu.VMEM((SUB,), jnp.int32),
            pltpu.VMEM((SUB,), jnp.int32),
            pltpu.VMEM((SUB,), jnp.int32),
            pltpu.VMEM((MAXE,), jnp.int32),
            pltpu.VMEM((MAXE,), jnp.int32),
            pltpu.VMEM((L,), jnp.int32),
            pltpu.VMEM((EB * K,), jnp.int32),
            pltpu.VMEM((EB * K,), jnp.float32),
            pltpu.VMEM((L,), jnp.int32),
            pltpu.VMEM((EB * K,), jnp.int32),
            pltpu.VMEM((EB * K, D), jnp.float32),
            pltpu.VMEM((SUB, D), jnp.float32),
            pltpu.VMEM((SUB, D), jnp.float32),
            pltpu.VMEM((BPW,), jnp.float32),
            pltpu.SemaphoreType.DMA,
        ],
    )
    return k(node_emb, rel_emb, head_i, tail_i, rel_i, deg_i, sim_f, nid_f)[0]
